# Initial kernel scaffold; baseline (speedup 1.0000x reference)
import jax
import jax.numpy as jnp
from jax.experimental import pallas as pl


def kernel(x, table, W, b):
    raise NotImplementedError("write your pallas kernel here")


# speedup vs baseline: 8.6929x; 8.6929x over previous
import functools
import jax
import jax.numpy as jnp
from jax import lax
from jax.experimental import pallas as pl
from jax.experimental.pallas import tpu as pltpu
from jax.experimental.pallas import tpu_sc as plsc

NUM_EMBEDDINGS = 1000000
EMBED_DIM = 32
BATCH = 16384
FIELDS = 26

BF = BATCH * FIELDS
CHUNK = 128
N_CHUNKS = BF // CHUNK
ROWS_PER_BLOCK = 8


def _make_sc_kernel():
    info = plsc.get_sparse_core_info()
    nw = info.num_cores * info.num_subcores
    chunks_per_w = N_CHUNKS // nw

    mesh = plsc.VectorSubcoreMesh(core_axis_name="c", subcore_axis_name="s")

    @functools.partial(
        pl.kernel,
        out_type=jax.ShapeDtypeStruct((BF, EMBED_DIM), jnp.float32),
        mesh=mesh,
        scratch_types=[
            pltpu.VMEM((chunks_per_w, CHUNK), jnp.int32),
            pltpu.VMEM((CHUNK, EMBED_DIM), jnp.float32),
            pltpu.VMEM((CHUNK, EMBED_DIM), jnp.float32),
            pltpu.VMEM((EMBED_DIM, EMBED_DIM), jnp.float32),
            pltpu.VMEM((EMBED_DIM,), jnp.float32),
            pltpu.SemaphoreType.DMA,
        ],
        compiler_params=pltpu.CompilerParams(use_tc_tiling_on_sc=False),
    )
    def emb_linear(x_hbm, table_hbm, wt_hbm, b_hbm, out_hbm,
                   idx_v, rows_v, out_v, wt_v, b_v, sem):
        wid = lax.axis_index("s") * info.num_cores + lax.axis_index("c")
        chunk0 = wid * chunks_per_w

        pltpu.sync_copy(x_hbm.at[pl.ds(chunk0, chunks_per_w)], idx_v)
        pltpu.sync_copy(wt_hbm, wt_v)
        pltpu.sync_copy(b_hbm, b_v)

        b0 = b_v[0:16]
        b1 = b_v[16:32]

        def do_chunk(c, _):
            pltpu.async_copy(table_hbm.at[idx_v.at[c]], rows_v, sem).wait()

            def do_block(rb, _):
                base = rb * ROWS_PER_BLOCK
                rows = [(rows_v[base + r, 0:16], rows_v[base + r, 16:32])
                        for r in range(ROWS_PER_BLOCK)]
                acc0 = [b0] * ROWS_PER_BLOCK
                acc1 = [b1] * ROWS_PER_BLOCK
                for d in range(EMBED_DIM):
                    w0 = wt_v[d, 0:16]
                    w1 = wt_v[d, 16:32]
                    for r in range(ROWS_PER_BLOCK):
                        s = rows[r][d // 16][d % 16]
                        acc0[r] = acc0[r] + s * w0
                        acc1[r] = acc1[r] + s * w1
                for r in range(ROWS_PER_BLOCK):
                    out_v[base + r, 0:16] = acc0[r]
                    out_v[base + r, 16:32] = acc1[r]
                return 0

            lax.fori_loop(0, CHUNK // ROWS_PER_BLOCK, do_block, 0)
            pltpu.sync_copy(out_v, out_hbm.at[pl.ds((chunk0 + c) * CHUNK, CHUNK)])
            return 0

        lax.fori_loop(0, chunks_per_w, do_chunk, 0)

    return emb_linear


_sc_kernel = _make_sc_kernel()


@jax.jit
def kernel(x, table, W, b):
    x2d = x.astype(jnp.int32).reshape(N_CHUNKS, CHUNK)
    wt = W.T
    out = _sc_kernel(x2d, table, wt, b)
    return out.reshape(BATCH, FIELDS, EMBED_DIM)
